# baseline (speedup 1.0000x reference)
import functools

import jax
import jax.numpy as jnp
from jax import lax
from jax.experimental import pallas as pl
from jax.experimental.pallas import tpu as pltpu
from jax.experimental.pallas import tpu_sc as plsc

_A = 3
_B = 16
_N = 128
_C = 15
_H = 64
_T = 4096
_LC = 5.0
_LN = 0.5

_NW = 32
_CHUNK = 128
_GROUP_ROWS = 8
_NGROUP = 12


def _softplus(x):
    return jnp.maximum(x, 0.0) + jnp.log(1.0 + jnp.exp(-jnp.abs(x)))


def _sigmoid(x):
    t = jnp.exp(-jnp.abs(x))
    return jnp.where(x >= 0, 1.0 / (1.0 + t), t / (1.0 + t))


def _prep_body(anc_ref, tgt_ref, idx_ref, tval_ref, keep_ref):
    cls = tgt_ref[:, :, 0]
    gx = tgt_ref[:, :, 1]
    gy = tgt_ref[:, :, 2]
    gw = tgt_ref[:, :, 3]
    gh = tgt_ref[:, :, 4]
    aw0 = anc_ref[0, 0]
    aw1 = anc_ref[1, 0]
    aw2 = anc_ref[2, 0]
    ah0 = anc_ref[0, 1]
    ah1 = anc_ref[1, 1]
    ah2 = anc_ref[2, 1]

    valid = cls != -1.0
    gi = jnp.clip(jnp.floor(gx * _T).astype(jnp.int32), 0, _T - 1)
    gj = jnp.clip(jnp.floor(gy * _H).astype(jnp.int32), 0, _H - 1)
    d0 = jnp.abs(aw0 - gw)
    d1 = jnp.abs(aw1 - gw)
    d2 = jnp.abs(aw2 - gw)
    b1 = d1 < d0
    bd = jnp.where(b1, d1, d0)
    b2 = d2 < bd
    best = jnp.where(b2, 2, jnp.where(b1, 1, 0)).astype(jnp.int32)
    awb = jnp.where(b2, aw2, jnp.where(b1, aw1, aw0))
    ahb = jnp.where(b2, ah2, jnp.where(b1, ah1, ah0))

    tval_ref[0] = gx * _T - gi.astype(jnp.float32)
    tval_ref[1] = gy * _H - gj.astype(jnp.float32)
    tval_ref[2] = jnp.log(gw / awb + 1e-16)
    tval_ref[3] = jnp.log(gh / ahb + 1e-16)

    cell = (best * _H + gj) * _T + gi

    cellT = jnp.transpose(cell)
    validT = jnp.transpose(valid.astype(jnp.int32)) != 0

    ii = lax.broadcasted_iota(jnp.int32, (_N, _N), 0)
    jj = lax.broadcasted_iota(jnp.int32, (_N, _N), 1)
    for b in range(_B):
        col = cellT[:, b:b + 1]
        vcol = validT[:, b:b + 1]
        row = cell[b:b + 1, :]
        beaten = (col == row) & (ii > jj) & vcol
        dup = jnp.any(beaten, axis=0, keepdims=True)
        keep_ref[b:b + 1, :] = jnp.where(
            valid[b:b + 1, :] & jnp.logical_not(dup), 1.0, 0.0)

    b_iota = lax.broadcasted_iota(jnp.int32, (_B, _N), 0)
    intile = (((gj >> 3) * (_T // 128) + (gi >> 7)) * 8 + (gj & 7)) * 128 + (gi & 127)
    base = (b_iota * _C + best * 5) * (_H * _T) + intile
    for k in range(5):
        idx_ref[k * _B:(k + 1) * _B, :] = base + k * (_H * _T)
    idx_ref[5 * _B:6 * _B, :] = jnp.zeros((_B, _N), jnp.int32)


@functools.lru_cache(maxsize=None)
def _make_prep():
    return pl.pallas_call(
        _prep_body,
        in_specs=[
            pl.BlockSpec(memory_space=pltpu.SMEM),
            pl.BlockSpec(memory_space=pltpu.VMEM),
        ],
        out_specs=(
            pl.BlockSpec(memory_space=pltpu.VMEM),
            pl.BlockSpec(memory_space=pltpu.VMEM),
            pl.BlockSpec(memory_space=pltpu.VMEM),
        ),
        out_shape=(
            jax.ShapeDtypeStruct((6 * _B, _N), jnp.int32),
            jax.ShapeDtypeStruct((4, _B, _N), jnp.float32),
            jax.ShapeDtypeStruct((_B, _N), jnp.float32),
        ),
    )


def _sc_gather_body(pred_hbm, idx_hbm, out_hbm, idx_v, rows_v, sem):
    wid = lax.axis_index("s") * 2 + lax.axis_index("c")

    @pl.when(wid < _NGROUP)
    def _():
        pltpu.sync_copy(idx_hbm.at[wid], idx_v)
        copies = [
            pltpu.async_copy(pred_hbm.at[idx_v.at[c]], rows_v.at[c], sem)
            for c in range(_GROUP_ROWS)
        ]
        for cp in copies:
            cp.wait()
        pltpu.sync_copy(rows_v, out_hbm.at[wid])


@functools.lru_cache(maxsize=None)
def _make_sc_gather():
    mesh = plsc.VectorSubcoreMesh(core_axis_name="c", subcore_axis_name="s")
    return functools.partial(
        pl.kernel,
        mesh=mesh,
        out_type=jax.ShapeDtypeStruct((_NGROUP, _GROUP_ROWS, _CHUNK), jnp.float32),
        scratch_types=[
            pltpu.VMEM((_GROUP_ROWS, _CHUNK), jnp.int32),
            pltpu.VMEM((_GROUP_ROWS, _CHUNK), jnp.float32),
            pltpu.SemaphoreType.DMA,
        ],
    )(_sc_gather_body)


def _dense_body(p0_ref, p1_ref, p2_ref, o_ref):
    i = pl.program_id(0)

    @pl.when(i == 0)
    def _():
        o_ref[0] = 0.0

    o_ref[0] += (jnp.sum(_softplus(p0_ref[...]))
                 + jnp.sum(_softplus(p1_ref[...]))
                 + jnp.sum(_softplus(p2_ref[...])))


@functools.lru_cache(maxsize=None)
def _make_dense():
    return pl.pallas_call(
        _dense_body,
        grid=(_B // 2,),
        in_specs=[
            pl.BlockSpec((2, 1, _H, _T), lambda i: (i, 4, 0, 0)),
            pl.BlockSpec((2, 1, _H, _T), lambda i: (i, 9, 0, 0)),
            pl.BlockSpec((2, 1, _H, _T), lambda i: (i, 14, 0, 0)),
        ],
        out_specs=pl.BlockSpec(memory_space=pltpu.SMEM),
        out_shape=jax.ShapeDtypeStruct((1,), jnp.float32),
    )


def _combine_body(s_ref, g_ref, tv_ref, keep_ref, o0_ref, o1_ref, o2_ref):
    keep = keep_ref[...]
    x = g_ref[0:16, :]
    y = g_ref[16:32, :]
    w = g_ref[32:48, :]
    h = g_ref[48:64, :]
    ob = g_ref[64:80, :]
    corr_obj = jnp.sum(keep * (_softplus(-ob) - _LN * _softplus(ob)))
    box = ((_sigmoid(x) - tv_ref[0]) ** 2 + (_sigmoid(y) - tv_ref[1]) ** 2
           + (w - tv_ref[2]) ** 2 + (h - tv_ref[3]) ** 2)
    loss_box = _LC * jnp.sum(keep * box)
    loss_obj = _LN * s_ref[0] + corr_obj
    o0_ref[...] = (loss_obj + loss_box) / _B
    o1_ref[...] = loss_obj / _B
    o2_ref[...] = loss_box / _B


@functools.lru_cache(maxsize=None)
def _make_combine():
    return pl.pallas_call(
        _combine_body,
        in_specs=[
            pl.BlockSpec(memory_space=pltpu.SMEM),
            pl.BlockSpec(memory_space=pltpu.VMEM),
            pl.BlockSpec(memory_space=pltpu.VMEM),
            pl.BlockSpec(memory_space=pltpu.VMEM),
        ],
        out_specs=(
            pl.BlockSpec(memory_space=pltpu.SMEM),
            pl.BlockSpec(memory_space=pltpu.SMEM),
            pl.BlockSpec(memory_space=pltpu.SMEM),
        ),
        out_shape=(
            jax.ShapeDtypeStruct((), jnp.float32),
            jax.ShapeDtypeStruct((), jnp.float32),
            jax.ShapeDtypeStruct((), jnp.float32),
        ),
    )


def kernel(predictions, targets, anchors):
    idx, tvals, keep = _make_prep()(anchors, targets)
    idx_grp = idx.reshape(_NGROUP, _GROUP_ROWS, _CHUNK)
    pred_phys = predictions.reshape(_B, _C, _H // 8, 8, _T // 128, 128)
    pred_phys = pred_phys.transpose(0, 1, 2, 4, 3, 5).reshape(-1)
    gathered = _make_sc_gather()(pred_phys, idx_grp)
    gathered = gathered.reshape(_NGROUP * _GROUP_ROWS, _CHUNK)
    s = _make_dense()(predictions, predictions, predictions)
    return _make_combine()(s, gathered, tvals, keep)


# speedup vs baseline: 1.1653x; 1.1653x over previous
import functools

import jax
import jax.numpy as jnp
from jax import lax
from jax.experimental import pallas as pl
from jax.experimental.pallas import tpu as pltpu
from jax.experimental.pallas import tpu_sc as plsc

_A = 3
_B = 16
_N = 128
_C = 15
_H = 64
_T = 4096
_LC = 5.0
_LN = 0.5

_NW = 32
_CHUNK = 128
_GROUP_ROWS = 8
_NGROUP = 12


def _softplus(x):
    return jnp.maximum(x, 0.0) + jnp.log(1.0 + jnp.exp(-jnp.abs(x)))


def _sigmoid(x):
    t = jnp.exp(-jnp.abs(x))
    return jnp.where(x >= 0, 1.0 / (1.0 + t), t / (1.0 + t))


def _prep_body(anc_ref, tgt_ref, tgtt_ref, idx_ref, tval_ref, keep_ref):
    cls = tgt_ref[0]
    gx = tgt_ref[1]
    gy = tgt_ref[2]
    gw = tgt_ref[3]
    gh = tgt_ref[4]
    aw0 = anc_ref[0, 0]
    aw1 = anc_ref[1, 0]
    aw2 = anc_ref[2, 0]
    ah0 = anc_ref[0, 1]
    ah1 = anc_ref[1, 1]
    ah2 = anc_ref[2, 1]

    valid = cls != -1.0
    gi = jnp.clip(jnp.floor(gx * _T).astype(jnp.int32), 0, _T - 1)
    gj = jnp.clip(jnp.floor(gy * _H).astype(jnp.int32), 0, _H - 1)
    d0 = jnp.abs(aw0 - gw)
    d1 = jnp.abs(aw1 - gw)
    d2 = jnp.abs(aw2 - gw)
    b1 = d1 < d0
    bd = jnp.where(b1, d1, d0)
    b2 = d2 < bd
    best = jnp.where(b2, 2, jnp.where(b1, 1, 0)).astype(jnp.int32)
    awb = jnp.where(b2, aw2, jnp.where(b1, aw1, aw0))
    ahb = jnp.where(b2, ah2, jnp.where(b1, ah1, ah0))

    tval_ref[0] = gx * _T - gi.astype(jnp.float32)
    tval_ref[1] = gy * _H - gj.astype(jnp.float32)
    tval_ref[2] = jnp.log(gw / awb + 1e-16)
    tval_ref[3] = jnp.log(gh / ahb + 1e-16)

    cell = (best * _H + gj) * _T + gi

    clsT = tgtt_ref[0]
    gxT = tgtt_ref[1]
    gyT = tgtt_ref[2]
    gwT = tgtt_ref[3]
    validT = clsT != -1.0
    giT = jnp.clip(jnp.floor(gxT * _T).astype(jnp.int32), 0, _T - 1)
    gjT = jnp.clip(jnp.floor(gyT * _H).astype(jnp.int32), 0, _H - 1)
    d0T = jnp.abs(aw0 - gwT)
    d1T = jnp.abs(aw1 - gwT)
    d2T = jnp.abs(aw2 - gwT)
    b1T = d1T < d0T
    bdT = jnp.where(b1T, d1T, d0T)
    b2T = d2T < bdT
    bestT = jnp.where(b2T, 2, jnp.where(b1T, 1, 0)).astype(jnp.int32)
    cellT = (bestT * _H + gjT) * _T + giT

    ii = lax.broadcasted_iota(jnp.int32, (_N, _N), 0)
    jj = lax.broadcasted_iota(jnp.int32, (_N, _N), 1)
    for b in range(_B):
        col = cellT[:, b:b + 1]
        vcol = validT[:, b:b + 1]
        row = cell[b:b + 1, :]
        beaten = (col == row) & (ii > jj) & vcol
        dup = jnp.any(beaten, axis=0, keepdims=True)
        keep_ref[b:b + 1, :] = jnp.where(
            valid[b:b + 1, :] & jnp.logical_not(dup), 1.0, 0.0)

    b_iota = lax.broadcasted_iota(jnp.int32, (_B, _N), 0)
    intile = (((gj >> 3) * (_T // 128) + (gi >> 7)) * 8 + (gj & 7)) * 128 + (gi & 127)
    base = (b_iota * _C + best * 5) * (_H * _T) + intile
    for k in range(5):
        idx_ref[k * _B:(k + 1) * _B, :] = base + k * (_H * _T)
    idx_ref[5 * _B:6 * _B, :] = jnp.zeros((_B, _N), jnp.int32)


@functools.lru_cache(maxsize=None)
def _make_prep():
    return pl.pallas_call(
        _prep_body,
        in_specs=[
            pl.BlockSpec(memory_space=pltpu.SMEM),
            pl.BlockSpec(memory_space=pltpu.VMEM),
            pl.BlockSpec(memory_space=pltpu.VMEM),
        ],
        out_specs=(
            pl.BlockSpec(memory_space=pltpu.VMEM),
            pl.BlockSpec(memory_space=pltpu.VMEM),
            pl.BlockSpec(memory_space=pltpu.VMEM),
        ),
        out_shape=(
            jax.ShapeDtypeStruct((6 * _B, _N), jnp.int32),
            jax.ShapeDtypeStruct((4, _B, _N), jnp.float32),
            jax.ShapeDtypeStruct((_B, _N), jnp.float32),
        ),
    )


def _sc_gather_body(pred_hbm, idx_hbm, out_hbm, idx_v, rows_v, sem):
    wid = lax.axis_index("s") * 2 + lax.axis_index("c")

    @pl.when(wid < _NGROUP)
    def _():
        pltpu.sync_copy(idx_hbm.at[wid], idx_v)
        copies = [
            pltpu.async_copy(pred_hbm.at[idx_v.at[c]], rows_v.at[c], sem)
            for c in range(_GROUP_ROWS)
        ]
        for cp in copies:
            cp.wait()
        pltpu.sync_copy(rows_v, out_hbm.at[wid])


@functools.lru_cache(maxsize=None)
def _make_sc_gather():
    mesh = plsc.VectorSubcoreMesh(core_axis_name="c", subcore_axis_name="s")
    return functools.partial(
        pl.kernel,
        mesh=mesh,
        out_type=jax.ShapeDtypeStruct((_NGROUP, _GROUP_ROWS, _CHUNK), jnp.float32),
        scratch_types=[
            pltpu.VMEM((_GROUP_ROWS, _CHUNK), jnp.int32),
            pltpu.VMEM((_GROUP_ROWS, _CHUNK), jnp.float32),
            pltpu.SemaphoreType.DMA,
        ],
    )(_sc_gather_body)


def _dense_body(p0_ref, p1_ref, p2_ref, o_ref):
    i = pl.program_id(0)

    @pl.when(i == 0)
    def _():
        o_ref[0] = 0.0

    o_ref[0] += (jnp.sum(_softplus(p0_ref[...]))
                 + jnp.sum(_softplus(p1_ref[...]))
                 + jnp.sum(_softplus(p2_ref[...])))


@functools.lru_cache(maxsize=None)
def _make_dense():
    return pl.pallas_call(
        _dense_body,
        grid=(_B // 2,),
        in_specs=[
            pl.BlockSpec((2, 1, _H, _T), lambda i: (i, 4, 0, 0)),
            pl.BlockSpec((2, 1, _H, _T), lambda i: (i, 9, 0, 0)),
            pl.BlockSpec((2, 1, _H, _T), lambda i: (i, 14, 0, 0)),
        ],
        out_specs=pl.BlockSpec(memory_space=pltpu.SMEM),
        out_shape=jax.ShapeDtypeStruct((1,), jnp.float32),
    )


def _combine_body(s_ref, g_ref, tv_ref, keep_ref, o0_ref, o1_ref, o2_ref):
    keep = keep_ref[...]
    x = g_ref[0:16, :]
    y = g_ref[16:32, :]
    w = g_ref[32:48, :]
    h = g_ref[48:64, :]
    ob = g_ref[64:80, :]
    corr_obj = jnp.sum(keep * (_softplus(-ob) - _LN * _softplus(ob)))
    box = ((_sigmoid(x) - tv_ref[0]) ** 2 + (_sigmoid(y) - tv_ref[1]) ** 2
           + (w - tv_ref[2]) ** 2 + (h - tv_ref[3]) ** 2)
    loss_box = _LC * jnp.sum(keep * box)
    loss_obj = _LN * s_ref[0] + corr_obj
    o0_ref[...] = (loss_obj + loss_box) / _B
    o1_ref[...] = loss_obj / _B
    o2_ref[...] = loss_box / _B


@functools.lru_cache(maxsize=None)
def _make_combine():
    return pl.pallas_call(
        _combine_body,
        in_specs=[
            pl.BlockSpec(memory_space=pltpu.SMEM),
            pl.BlockSpec(memory_space=pltpu.VMEM),
            pl.BlockSpec(memory_space=pltpu.VMEM),
            pl.BlockSpec(memory_space=pltpu.VMEM),
        ],
        out_specs=(
            pl.BlockSpec(memory_space=pltpu.SMEM),
            pl.BlockSpec(memory_space=pltpu.SMEM),
            pl.BlockSpec(memory_space=pltpu.SMEM),
        ),
        out_shape=(
            jax.ShapeDtypeStruct((), jnp.float32),
            jax.ShapeDtypeStruct((), jnp.float32),
            jax.ShapeDtypeStruct((), jnp.float32),
        ),
    )


def kernel(predictions, targets, anchors):
    tgt = jnp.transpose(targets, (2, 0, 1))
    tgtt = jnp.transpose(targets, (2, 1, 0))
    idx, tvals, keep = _make_prep()(anchors, tgt, tgtt)
    idx_grp = idx.reshape(_NGROUP, _GROUP_ROWS, _CHUNK)
    pred_phys = predictions.reshape(_B, _C, _H // 8, 8, _T // 128, 128)
    pred_phys = pred_phys.transpose(0, 1, 2, 4, 3, 5).reshape(-1)
    gathered = _make_sc_gather()(pred_phys, idx_grp)
    gathered = gathered.reshape(_NGROUP * _GROUP_ROWS, _CHUNK)
    s = _make_dense()(predictions, predictions, predictions)
    return _make_combine()(s, gathered, tvals, keep)
